# baseline (speedup 1.0000x reference)
import functools

import jax
import jax.numpy as jnp
from jax import lax
from jax.experimental import pallas as pl
from jax.experimental.pallas import tpu as pltpu
from jax.experimental.pallas import tpu_sc as plsc

B, L, H, W = 2, 8, 128, 128
CI, CF = 32, 2
CB = CI + CF
CC = 48
NPX = H * W
R = B * L * NPX
NC, NS, LN = 2, 16, 16
NW = NC * NS
P = 128
GROUPS = P // LN
CPS = NPX // P


def _floorf(x):
    i = x.astype(jnp.int32)
    f = i.astype(jnp.float32)
    return jnp.where(f > x, f - 1.0, f)


def _make_round(s):
    nsl = B * (L - s)
    pw = nsl * CPS // NW

    mesh = plsc.VectorSubcoreMesh(
        core_axis_name="c", subcore_axis_name="s",
        num_cores=NC, num_subcores=NS)

    scratch = (
        [pltpu.VMEM((P, CC), jnp.float32)] * 3
        + [pltpu.VMEM((P, CC), jnp.float32)] * 3
        + [pltpu.VMEM((P, CC), jnp.float32)] * 12
        + [pltpu.VMEM((P,), jnp.int32)] * 12
        + [pltpu.VMEM((P,), jnp.float32)] * 12
        + [pltpu.SemaphoreType.DMA] * 18
    )

    @functools.partial(
        pl.kernel,
        out_type=jax.ShapeDtypeStruct((R, CC), jnp.float32),
        mesh=mesh,
        scratch_types=scratch,
        compiler_params=pltpu.CompilerParams(
            needs_layout_passes=False, use_tc_tiling_on_sc=False),
    )
    def round_kernel(state_in, state_out, *scr):
        cur = scr[0:3]
        oro = scr[3:6]
        taps = [scr[6 + 4 * q:10 + 4 * q] for q in range(3)]
        ib = [scr[18 + 4 * q:22 + 4 * q] for q in range(3)]
        wb = [scr[30 + 4 * q:34 + 4 * q] for q in range(3)]
        semc = scr[42:45]
        semo = scr[45:48]
        semg = [scr[48 + 4 * q:52 + 4 * q] for q in range(3)]

        wid = lax.axis_index("s") * NC + lax.axis_index("c")

        for b in range(B):
            for l in range(s):
                base = (b * L + l) * NPX + wid * (NPX // NW)
                for t in range(4):
                    pltpu.async_copy(state_in.at[pl.ds(base + t * P, P)],
                                     taps[0][t], semg[0][t])
                for t in range(4):
                    pltpu.make_async_copy(state_in.at[pl.ds(0, P)],
                                          taps[0][t], semg[0][t]).wait()
                    pltpu.async_copy(taps[0][t],
                                     state_out.at[pl.ds(base + t * P, P)],
                                     semg[0][t])
                for t in range(4):
                    pltpu.make_async_copy(state_in.at[pl.ds(0, P)],
                                          taps[0][t], semg[0][t]).wait()

        def coords(k):
            g = k * NW + wid
            sl = g >> 7
            p0 = (g - sl * CPS) * P
            bb = (sl >= (L - s)).astype(jnp.int32)
            ll = sl - bb * (L - s) + s
            slice_cur = bb * L + ll
            return slice_cur * NPX + p0, (slice_cur - s) * NPX, p0

        def fire_cur(k, q):
            rc, _, _ = coords(k)
            pltpu.async_copy(state_in.at[pl.ds(rc, P)], cur[q], semc[q])

        def idx_pass(k, q):
            _, prev_base, p0 = coords(k)

            @plsc.parallel_loop(0, GROUPS)
            def _idx(j):
                lane = lax.iota(jnp.int32, LN)
                loc = j * LN + lane
                pix = p0 + loc
                wi = pix & (W - 1)
                hi = pix >> 7
                fx = plsc.load_gather(cur[q], [loc, jnp.full((LN,), CI, jnp.int32)])
                fy = plsc.load_gather(cur[q], [loc, jnp.full((LN,), CI + 1, jnp.int32)])
                gx = wi.astype(jnp.float32) * (2.0 / W) + (1.0 / W - 1.0)
                gy = hi.astype(jnp.float32) * (2.0 / H) + (1.0 / H - 1.0)
                tx = gx + fx + 1.0
                tx = tx - 2.0 * _floorf(tx * 0.5)
                rx = tx * (W * 0.5) - 0.5
                ry = (gy + fy + 1.0) * (H * 0.5) - 0.5
                x0 = _floorf(rx)
                y0 = _floorf(ry)
                wx1 = rx - x0
                wx0 = 1.0 - wx1
                wy1 = ry - y0
                wy0 = 1.0 - wy1
                ix0 = x0.astype(jnp.int32)
                iy0 = y0.astype(jnp.int32)
                ix1 = ix0 + 1
                iy1 = iy0 + 1

                def tap(iy, ix, wgt, t):
                    valid = (ix >= 0) & (ix < W) & (iy >= 0) & (iy < H)
                    idx = (prev_base
                           + jnp.clip(iy, 0, H - 1) * W
                           + jnp.clip(ix, 0, W - 1))
                    ib[q][t][pl.ds(j * LN, LN)] = idx
                    wb[q][t][pl.ds(j * LN, LN)] = jnp.where(valid, wgt, 0.0)

                tap(iy0, ix0, wx0 * wy0, 0)
                tap(iy0, ix1, wx1 * wy0, 1)
                tap(iy1, ix0, wx0 * wy1, 2)
                tap(iy1, ix1, wx1 * wy1, 3)

        def blend(k, q):
            rc, _, _ = coords(k)
            for t in range(4):
                pltpu.make_async_copy(state_in.at[pl.ds(0, P)],
                                      taps[q][t], semg[q][t]).wait()
            @plsc.parallel_loop(0, GROUPS, unroll=2)
            def _blend(j):
                lane = lax.iota(jnp.int32, LN)
                rows = j * LN + lane
                wv0 = wb[q][0][pl.ds(j * LN, LN)]
                wv1 = wb[q][1][pl.ds(j * LN, LN)]
                wv2 = wb[q][2][pl.ds(j * LN, LN)]
                wv3 = wb[q][3][pl.ds(j * LN, LN)]
                for c in range(CB):
                    cc = jnp.full((LN,), c, jnp.int32)
                    a01 = (wv0 * plsc.load_gather(taps[q][0], [rows, cc])
                           + wv1 * plsc.load_gather(taps[q][1], [rows, cc]))
                    a23 = (wv2 * plsc.load_gather(taps[q][2], [rows, cc])
                           + wv3 * plsc.load_gather(taps[q][3], [rows, cc]))
                    res = plsc.load_gather(cur[q], [rows, cc])
                    plsc.store_scatter(oro[q], [rows, cc], res + a01 + a23)

            pltpu.async_copy(oro[q], state_out.at[pl.ds(rc, P)], semo[q])

        @pl.when(pw > 0)
        def _():
            fire_cur(0, 0)

        nbody = (pw + 4) // 3

        @pl.loop(0, nbody)
        def _outer(kk):
            for u in range(3):
                k = kk * 3 + u
                p1 = (u + 1) % 3
                p2 = (u + 2) % 3

                @pl.when(k < pw)
                def _(k=k, u=u):
                    pltpu.make_async_copy(state_in.at[pl.ds(0, P)],
                                          cur[u], semc[u]).wait()
                    idx_pass(k, u)
                    _, _, _ = coords(k)
                    for t in range(4):
                        pltpu.async_copy(state_in.at[ib[u][t]],
                                         taps[u][t], semg[u][t])

                @pl.when((k >= 2) & (k <= pw + 1))
                def _(k=k, p1=p1):
                    pltpu.make_async_copy(state_in.at[pl.ds(0, P)],
                                          oro[p1], semo[p1]).wait()

                @pl.when(k <= pw - 2)
                def _(k=k, p1=p1):
                    fire_cur(k + 1, p1)

                @pl.when((k >= 1) & (k <= pw))
                def _(k=k, p2=p2):
                    blend(k - 1, p2)

    return round_kernel


_ROUNDS = {s: _make_round(s) for s in (1, 2, 4)}


def kernel(flows, images):
    fl = flows.astype(jnp.float32)
    im = images.astype(jnp.float32)
    imgs_px = jnp.transpose(im.reshape(B, L, CI, NPX), (0, 1, 3, 2))
    flows_px = jnp.transpose(fl.reshape(B, L, CF, NPX), (0, 1, 3, 2))
    pad = jnp.zeros((B, L, NPX, CC - CB), jnp.float32)
    state = jnp.concatenate([imgs_px, flows_px, pad], axis=-1).reshape(R, CC)
    for s in (1, 2, 4):
        state = _ROUNDS[s](state)
    out = state.reshape(B, L, NPX, CC)[..., :CI]
    return jnp.transpose(out, (0, 1, 3, 2)).reshape(B, L, CI, H, W)


# speedup vs baseline: 1.4449x; 1.4449x over previous
import functools

import jax
import jax.numpy as jnp
from jax import lax
from jax.experimental import pallas as pl
from jax.experimental.pallas import tpu as pltpu
from jax.experimental.pallas import tpu_sc as plsc

B, L, H, W = 2, 8, 128, 128
CI, CF = 32, 2
CIW = CI // 2
CC = 24
NPX = H * W
R = B * L * NPX
NC, NS, LN = 2, 16, 16
NW = NC * NS
P = 128
GROUPS = P // LN
CPS = NPX // P


def _floorf(x):
    i = x.astype(jnp.int32)
    f = i.astype(jnp.float32)
    return jnp.where(f > x, f - 1.0, f)


def _make_round(s):
    nsl = B * (L - s)
    pw = nsl * CPS // NW

    mesh = plsc.VectorSubcoreMesh(
        core_axis_name="c", subcore_axis_name="s",
        num_cores=NC, num_subcores=NS)

    scratch = (
        [pltpu.VMEM((P, CC), jnp.int32)] * 3
        + [pltpu.VMEM((P, CC), jnp.int32)] * 3
        + [pltpu.VMEM((P, CC), jnp.int32)] * 12
        + [pltpu.VMEM((P,), jnp.int32)] * 12
        + [pltpu.VMEM((P,), jnp.float32)] * 12
        + [pltpu.SemaphoreType.DMA] * 18
    )

    @functools.partial(
        pl.kernel,
        out_type=jax.ShapeDtypeStruct((R, CC), jnp.int32),
        mesh=mesh,
        scratch_types=scratch,
        compiler_params=pltpu.CompilerParams(
            needs_layout_passes=False, use_tc_tiling_on_sc=False),
    )
    def round_kernel(state_in, state_out, *scr):
        cur = scr[0:3]
        oro = scr[3:6]
        taps = [scr[6 + 4 * q:10 + 4 * q] for q in range(3)]
        ib = [scr[18 + 4 * q:22 + 4 * q] for q in range(3)]
        wb = [scr[30 + 4 * q:34 + 4 * q] for q in range(3)]
        semc = scr[42:45]
        semo = scr[45:48]
        semg = [scr[48 + 4 * q:52 + 4 * q] for q in range(3)]

        wid = lax.axis_index("s") * NC + lax.axis_index("c")

        for b in range(B):
            for l in range(s):
                base = (b * L + l) * NPX + wid * (NPX // NW)
                for t in range(4):
                    pltpu.async_copy(state_in.at[pl.ds(base + t * P, P)],
                                     taps[0][t], semg[0][t])
                for t in range(4):
                    pltpu.make_async_copy(state_in.at[pl.ds(0, P)],
                                          taps[0][t], semg[0][t]).wait()
                    pltpu.async_copy(taps[0][t],
                                     state_out.at[pl.ds(base + t * P, P)],
                                     semg[0][t])
                for t in range(4):
                    pltpu.make_async_copy(state_in.at[pl.ds(0, P)],
                                          taps[0][t], semg[0][t]).wait()

        def coords(k):
            g = k * NW + wid
            sl = g >> 7
            p0 = (g - sl * CPS) * P
            bb = (sl >= (L - s)).astype(jnp.int32)
            ll = sl - bb * (L - s) + s
            slice_cur = bb * L + ll
            return slice_cur * NPX + p0, (slice_cur - s) * NPX, p0

        def fire_cur(k, q):
            rc, _, _ = coords(k)
            pltpu.async_copy(state_in.at[pl.ds(rc, P)], cur[q], semc[q])

        def idx_pass(k, q):
            _, prev_base, p0 = coords(k)

            @plsc.parallel_loop(0, GROUPS)
            def _idx(j):
                lane = lax.iota(jnp.int32, LN)
                loc = j * LN + lane
                pix = p0 + loc
                wi = pix & (W - 1)
                hi = pix >> 7
                fx = plsc.bitcast(
                    plsc.load_gather(cur[q], [loc, jnp.full((LN,), CIW, jnp.int32)]),
                    jnp.float32)
                fy = plsc.bitcast(
                    plsc.load_gather(cur[q], [loc, jnp.full((LN,), CIW + 1, jnp.int32)]),
                    jnp.float32)
                gx = wi.astype(jnp.float32) * (2.0 / W) + (1.0 / W - 1.0)
                gy = hi.astype(jnp.float32) * (2.0 / H) + (1.0 / H - 1.0)
                tx = gx + fx + 1.0
                tx = tx - 2.0 * _floorf(tx * 0.5)
                rx = tx * (W * 0.5) - 0.5
                ry = (gy + fy + 1.0) * (H * 0.5) - 0.5
                x0 = _floorf(rx)
                y0 = _floorf(ry)
                wx1 = rx - x0
                wx0 = 1.0 - wx1
                wy1 = ry - y0
                wy0 = 1.0 - wy1
                ix0 = x0.astype(jnp.int32)
                iy0 = y0.astype(jnp.int32)
                ix1 = ix0 + 1
                iy1 = iy0 + 1

                def tap(iy, ix, wgt, t):
                    valid = (ix >= 0) & (ix < W) & (iy >= 0) & (iy < H)
                    idx = (prev_base
                           + jnp.clip(iy, 0, H - 1) * W
                           + jnp.clip(ix, 0, W - 1))
                    ib[q][t][pl.ds(j * LN, LN)] = idx
                    wb[q][t][pl.ds(j * LN, LN)] = jnp.where(valid, wgt, 0.0)

                tap(iy0, ix0, wx0 * wy0, 0)
                tap(iy0, ix1, wx1 * wy0, 1)
                tap(iy1, ix0, wx0 * wy1, 2)
                tap(iy1, ix1, wx1 * wy1, 3)

        def blend(k, q):
            rc, _, _ = coords(k)
            for t in range(4):
                pltpu.make_async_copy(state_in.at[pl.ds(0, P)],
                                      taps[q][t], semg[q][t]).wait()

            @plsc.parallel_loop(0, GROUPS, unroll=2)
            def _blend(j):
                lane = lax.iota(jnp.int32, LN)
                rows = j * LN + lane
                wv0 = wb[q][0][pl.ds(j * LN, LN)]
                wv1 = wb[q][1][pl.ds(j * LN, LN)]
                wv2 = wb[q][2][pl.ds(j * LN, LN)]
                wv3 = wb[q][3][pl.ds(j * LN, LN)]
                wp0 = plsc.pack(wv0, wv0, format=plsc.PackFormat.INTERLEAVED)
                wp1 = plsc.pack(wv1, wv1, format=plsc.PackFormat.INTERLEAVED)
                wp2 = plsc.pack(wv2, wv2, format=plsc.PackFormat.INTERLEAVED)
                wp3 = plsc.pack(wv3, wv3, format=plsc.PackFormat.INTERLEAVED)
                for c in range(CIW):
                    cc = jnp.full((LN,), c, jnp.int32)
                    g0 = plsc.bitcast(plsc.load_gather(taps[q][0], [rows, cc]),
                                      jnp.bfloat16)
                    g1 = plsc.bitcast(plsc.load_gather(taps[q][1], [rows, cc]),
                                      jnp.bfloat16)
                    g2 = plsc.bitcast(plsc.load_gather(taps[q][2], [rows, cc]),
                                      jnp.bfloat16)
                    g3 = plsc.bitcast(plsc.load_gather(taps[q][3], [rows, cc]),
                                      jnp.bfloat16)
                    res = plsc.bitcast(plsc.load_gather(cur[q], [rows, cc]),
                                       jnp.bfloat16)
                    acc = (res + (wp0 * g0 + wp1 * g1)) + (wp2 * g2 + wp3 * g3)
                    plsc.store_scatter(oro[q], [rows, cc],
                                       plsc.bitcast(acc, jnp.int32))
                for c in (CIW, CIW + 1):
                    cc = jnp.full((LN,), c, jnp.int32)
                    g0 = plsc.bitcast(plsc.load_gather(taps[q][0], [rows, cc]),
                                      jnp.float32)
                    g1 = plsc.bitcast(plsc.load_gather(taps[q][1], [rows, cc]),
                                      jnp.float32)
                    g2 = plsc.bitcast(plsc.load_gather(taps[q][2], [rows, cc]),
                                      jnp.float32)
                    g3 = plsc.bitcast(plsc.load_gather(taps[q][3], [rows, cc]),
                                      jnp.float32)
                    res = plsc.bitcast(plsc.load_gather(cur[q], [rows, cc]),
                                       jnp.float32)
                    acc = res + (wv0 * g0 + wv1 * g1) + (wv2 * g2 + wv3 * g3)
                    plsc.store_scatter(oro[q], [rows, cc],
                                       plsc.bitcast(acc, jnp.int32))

            pltpu.async_copy(oro[q], state_out.at[pl.ds(rc, P)], semo[q])

        @pl.when(pw > 0)
        def _():
            fire_cur(0, 0)

        nbody = (pw + 4) // 3

        @pl.loop(0, nbody)
        def _outer(kk):
            for u in range(3):
                k = kk * 3 + u
                p1 = (u + 1) % 3
                p2 = (u + 2) % 3

                @pl.when(k < pw)
                def _(k=k, u=u):
                    pltpu.make_async_copy(state_in.at[pl.ds(0, P)],
                                          cur[u], semc[u]).wait()
                    idx_pass(k, u)
                    for t in range(4):
                        pltpu.async_copy(state_in.at[ib[u][t]],
                                         taps[u][t], semg[u][t])

                @pl.when((k >= 2) & (k <= pw + 1))
                def _(k=k, p1=p1):
                    pltpu.make_async_copy(state_in.at[pl.ds(0, P)],
                                          oro[p1], semo[p1]).wait()

                @pl.when(k <= pw - 2)
                def _(k=k, p1=p1):
                    fire_cur(k + 1, p1)

                @pl.when((k >= 1) & (k <= pw))
                def _(k=k, p2=p2):
                    blend(k - 1, p2)

    return round_kernel


_ROUNDS = {s: _make_round(s) for s in (1, 2, 4)}


def kernel(flows, images):
    fl = flows.astype(jnp.float32)
    im = images.astype(jnp.float32)
    imgs_px = jnp.transpose(im.reshape(B, L, CI, NPX), (0, 1, 3, 2))
    img_words = lax.bitcast_convert_type(
        imgs_px.astype(jnp.bfloat16).reshape(B, L, NPX, CIW, 2), jnp.int32)
    flows_px = jnp.transpose(fl.reshape(B, L, CF, NPX), (0, 1, 3, 2))
    flow_words = lax.bitcast_convert_type(flows_px, jnp.int32)
    pad = jnp.zeros((B, L, NPX, CC - CIW - CF), jnp.int32)
    state = jnp.concatenate([img_words, flow_words, pad], axis=-1).reshape(R, CC)
    for s in (1, 2, 4):
        state = _ROUNDS[s](state)
    out_bf = lax.bitcast_convert_type(
        state.reshape(B, L, NPX, CC)[..., :CIW], jnp.bfloat16)
    out = out_bf.reshape(B, L, NPX, CI).astype(jnp.float32)
    return jnp.transpose(out, (0, 1, 3, 2)).reshape(B, L, CI, H, W)


# speedup vs baseline: 1.4614x; 1.0114x over previous
import functools

import jax
import jax.numpy as jnp
from jax import lax
from jax.experimental import pallas as pl
from jax.experimental.pallas import tpu as pltpu
from jax.experimental.pallas import tpu_sc as plsc

B, L, H, W = 2, 8, 128, 128
CI, CF = 32, 2
CIW = CI // 2
CC = 24
NPX = H * W
R = B * L * NPX
NC, NS, LN = 2, 16, 16
P = 128
GROUPS = P // LN
CPS = NPX // P
PWS = (56, 48, 32)
NBODYS = (20, 17, 12)
SS = (1, 2, 4)


def _floorf(x):
    i = x.astype(jnp.int32)
    f = i.astype(jnp.float32)
    return jnp.where(f > x, f - 1.0, f)


def _make_kernel():
    mesh = plsc.VectorSubcoreMesh(
        core_axis_name="c", subcore_axis_name="s",
        num_cores=NC, num_subcores=NS)

    st = jax.ShapeDtypeStruct((R, CC), jnp.int32)
    scratch = (
        [pltpu.VMEM((P, CC), jnp.int32)] * 3
        + [pltpu.VMEM((P, CC), jnp.int32)] * 3
        + [pltpu.VMEM((P, CC), jnp.int32)] * 12
        + [pltpu.VMEM((P,), jnp.int32)] * 12
        + [pltpu.VMEM((P,), jnp.float32)] * 12
        + [pltpu.SemaphoreType.DMA] * 18
    )

    @functools.partial(
        pl.kernel,
        out_type=(st, st, st),
        mesh=mesh,
        scratch_types=scratch,
        compiler_params=pltpu.CompilerParams(
            needs_layout_passes=False, use_tc_tiling_on_sc=False),
    )
    def pscan_kernel(state0, out, st1, st2, *scr):
        cur = scr[0:3]
        oro = scr[3:6]
        taps = [scr[6 + 4 * q:10 + 4 * q] for q in range(3)]
        ib = [scr[18 + 4 * q:22 + 4 * q] for q in range(3)]
        wb = [scr[30 + 4 * q:34 + 4 * q] for q in range(3)]
        semc = scr[42:45]
        semo = scr[45:48]
        semg = [scr[48 + 4 * q:52 + 4 * q] for q in range(3)]

        core = lax.axis_index("c")
        sid = lax.axis_index("s")
        rsd = ((state0, st1), (st1, st2), (st2, out))

        @pl.loop(0, 3)
        def _round(rnd):
            s = jnp.where(rnd == 0, SS[0], jnp.where(rnd == 1, SS[1], SS[2]))
            pw = jnp.where(rnd == 0, PWS[0], jnp.where(rnd == 1, PWS[1], PWS[2]))
            nbody = jnp.where(rnd == 0, NBODYS[0],
                              jnp.where(rnd == 1, NBODYS[1], NBODYS[2]))

            def fire_src(rc, buf, sem):
                for r, (sr, _) in enumerate(rsd):
                    @pl.when(rnd == r)
                    def _(sr=sr):
                        pltpu.async_copy(sr.at[pl.ds(rc, P)], buf, sem)

            def fire_gather(idxref, buf, sem):
                for r, (sr, _) in enumerate(rsd):
                    @pl.when(rnd == r)
                    def _(sr=sr):
                        pltpu.async_copy(sr.at[idxref], buf, sem)

            def fire_store(buf, rc, sem):
                for r, (_, dsr) in enumerate(rsd):
                    @pl.when(rnd == r)
                    def _(dsr=dsr):
                        pltpu.async_copy(buf, dsr.at[pl.ds(rc, P)], sem)

            def wait(buf, sem):
                pltpu.make_async_copy(state0.at[pl.ds(0, P)], buf, sem).wait()

            @pl.loop(0, s)
            def _prefix(l):
                base = (core * L + l) * NPX + sid * (NPX // NS)
                for half in range(2):
                    for t in range(4):
                        fire_src(base + (half * 4 + t) * P, taps[0][t],
                                 semg[0][t])
                    for t in range(4):
                        wait(taps[0][t], semg[0][t])
                        fire_store(taps[0][t], base + (half * 4 + t) * P,
                                   semg[0][t])
                    for t in range(4):
                        wait(taps[0][t], semg[0][t])

            def coords(k):
                g = k * NS + sid
                sl = g >> 7
                p0 = (g - sl * CPS) * P
                slice_cur = core * L + sl + s
                return slice_cur * NPX + p0, (slice_cur - s) * NPX, p0

            def fire_cur(k, q):
                rc, _, _ = coords(k)
                fire_src(rc, cur[q], semc[q])

            def idx_pass(k, q):
                _, prev_base, p0 = coords(k)

                @plsc.parallel_loop(0, GROUPS)
                def _idx(j):
                    lane = lax.iota(jnp.int32, LN)
                    loc = j * LN + lane
                    pix = p0 + loc
                    wi = pix & (W - 1)
                    hi = pix >> 7
                    fx = plsc.bitcast(
                        plsc.load_gather(
                            cur[q], [loc, jnp.full((LN,), CIW, jnp.int32)]),
                        jnp.float32)
                    fy = plsc.bitcast(
                        plsc.load_gather(
                            cur[q], [loc, jnp.full((LN,), CIW + 1, jnp.int32)]),
                        jnp.float32)
                    gx = wi.astype(jnp.float32) * (2.0 / W) + (1.0 / W - 1.0)
                    gy = hi.astype(jnp.float32) * (2.0 / H) + (1.0 / H - 1.0)
                    tx = gx + fx + 1.0
                    tx = tx - 2.0 * _floorf(tx * 0.5)
                    rx = tx * (W * 0.5) - 0.5
                    ry = (gy + fy + 1.0) * (H * 0.5) - 0.5
                    x0 = _floorf(rx)
                    y0 = _floorf(ry)
                    wx1 = rx - x0
                    wx0 = 1.0 - wx1
                    wy1 = ry - y0
                    wy0 = 1.0 - wy1
                    ix0 = x0.astype(jnp.int32)
                    iy0 = y0.astype(jnp.int32)
                    ix1 = ix0 + 1
                    iy1 = iy0 + 1

                    def tap(iy, ix, wgt, t):
                        valid = (ix >= 0) & (ix < W) & (iy >= 0) & (iy < H)
                        idx = (prev_base
                               + jnp.clip(iy, 0, H - 1) * W
                               + jnp.clip(ix, 0, W - 1))
                        ib[q][t][pl.ds(j * LN, LN)] = idx
                        wb[q][t][pl.ds(j * LN, LN)] = jnp.where(valid, wgt, 0.0)

                    tap(iy0, ix0, wx0 * wy0, 0)
                    tap(iy0, ix1, wx1 * wy0, 1)
                    tap(iy1, ix0, wx0 * wy1, 2)
                    tap(iy1, ix1, wx1 * wy1, 3)

            def blend(k, q):
                rc, _, _ = coords(k)
                for t in range(4):
                    wait(taps[q][t], semg[q][t])

                @plsc.parallel_loop(0, GROUPS, unroll=2)
                def _blend(j):
                    lane = lax.iota(jnp.int32, LN)
                    rows = j * LN + lane
                    wv0 = wb[q][0][pl.ds(j * LN, LN)]
                    wv1 = wb[q][1][pl.ds(j * LN, LN)]
                    wv2 = wb[q][2][pl.ds(j * LN, LN)]
                    wv3 = wb[q][3][pl.ds(j * LN, LN)]
                    wp0 = plsc.pack(wv0, wv0, format=plsc.PackFormat.INTERLEAVED)
                    wp1 = plsc.pack(wv1, wv1, format=plsc.PackFormat.INTERLEAVED)
                    wp2 = plsc.pack(wv2, wv2, format=plsc.PackFormat.INTERLEAVED)
                    wp3 = plsc.pack(wv3, wv3, format=plsc.PackFormat.INTERLEAVED)
                    for c in range(CIW):
                        cc = jnp.full((LN,), c, jnp.int32)
                        g0 = plsc.bitcast(
                            plsc.load_gather(taps[q][0], [rows, cc]), jnp.bfloat16)
                        g1 = plsc.bitcast(
                            plsc.load_gather(taps[q][1], [rows, cc]), jnp.bfloat16)
                        g2 = plsc.bitcast(
                            plsc.load_gather(taps[q][2], [rows, cc]), jnp.bfloat16)
                        g3 = plsc.bitcast(
                            plsc.load_gather(taps[q][3], [rows, cc]), jnp.bfloat16)
                        res = plsc.bitcast(
                            plsc.load_gather(cur[q], [rows, cc]), jnp.bfloat16)
                        acc = (res + (wp0 * g0 + wp1 * g1)) + (wp2 * g2 + wp3 * g3)
                        plsc.store_scatter(oro[q], [rows, cc],
                                           plsc.bitcast(acc, jnp.int32))
                    for c in (CIW, CIW + 1):
                        cc = jnp.full((LN,), c, jnp.int32)
                        g0 = plsc.bitcast(
                            plsc.load_gather(taps[q][0], [rows, cc]), jnp.float32)
                        g1 = plsc.bitcast(
                            plsc.load_gather(taps[q][1], [rows, cc]), jnp.float32)
                        g2 = plsc.bitcast(
                            plsc.load_gather(taps[q][2], [rows, cc]), jnp.float32)
                        g3 = plsc.bitcast(
                            plsc.load_gather(taps[q][3], [rows, cc]), jnp.float32)
                        res = plsc.bitcast(
                            plsc.load_gather(cur[q], [rows, cc]), jnp.float32)
                        acc = res + (wv0 * g0 + wv1 * g1) + (wv2 * g2 + wv3 * g3)
                        plsc.store_scatter(oro[q], [rows, cc],
                                           plsc.bitcast(acc, jnp.int32))

                fire_store(oro[q], rc, semo[q])

            fire_cur(0, 0)

            @pl.loop(0, nbody)
            def _outer(kk):
                for u in range(3):
                    k = kk * 3 + u
                    p1 = (u + 1) % 3
                    p2 = (u + 2) % 3

                    @pl.when(k < pw)
                    def _(k=k, u=u):
                        wait(cur[u], semc[u])
                        idx_pass(k, u)
                        for t in range(4):
                            fire_gather(ib[u][t], taps[u][t], semg[u][t])

                    @pl.when((k >= 2) & (k <= pw + 1))
                    def _(k=k, p1=p1):
                        wait(oro[p1], semo[p1])

                    @pl.when(k <= pw - 2)
                    def _(k=k, p1=p1):
                        fire_cur(k + 1, p1)

                    @pl.when((k >= 1) & (k <= pw))
                    def _(k=k, p2=p2):
                        blend(k - 1, p2)

            plsc.subcore_barrier()

    return pscan_kernel


_KERNEL = _make_kernel()


def kernel(flows, images):
    fl = flows.astype(jnp.float32)
    im = images.astype(jnp.float32)
    imgs_px = jnp.transpose(im.reshape(B, L, CI, NPX), (0, 1, 3, 2))
    img_words = lax.bitcast_convert_type(
        imgs_px.astype(jnp.bfloat16).reshape(B, L, NPX, CIW, 2), jnp.int32)
    flows_px = jnp.transpose(fl.reshape(B, L, CF, NPX), (0, 1, 3, 2))
    flow_words = lax.bitcast_convert_type(flows_px, jnp.int32)
    pad = jnp.zeros((B, L, NPX, CC - CIW - CF), jnp.int32)
    state = jnp.concatenate([img_words, flow_words, pad], axis=-1).reshape(R, CC)
    state, _, _ = _KERNEL(state)
    out_bf = lax.bitcast_convert_type(
        state.reshape(B, L, NPX, CC)[..., :CIW], jnp.bfloat16)
    out = out_bf.reshape(B, L, NPX, CI).astype(jnp.float32)
    return jnp.transpose(out, (0, 1, 3, 2)).reshape(B, L, CI, H, W)


# speedup vs baseline: 2.0308x; 1.3896x over previous
import functools

import jax
import jax.numpy as jnp
from jax import lax
from jax.experimental import pallas as pl
from jax.experimental.pallas import tpu as pltpu
from jax.experimental.pallas import tpu_sc as plsc

B, L, H, W = 2, 8, 128, 128
CI, CF = 32, 2
CIW = CI // 2
CC = 24
NPX = H * W
R = B * L * NPX
NC, NS, LN = 2, 16, 16
P = 128
GROUPS = P // LN
CPS = NPX // P
PWS = (56, 48, 32)
NBODYS = (20, 17, 12)
SS = (1, 2, 4)


def _floorf(x):
    i = x.astype(jnp.int32)
    f = i.astype(jnp.float32)
    return jnp.where(f > x, f - 1.0, f)


def _make_kernel():
    mesh = plsc.VectorSubcoreMesh(
        core_axis_name="c", subcore_axis_name="s",
        num_cores=NC, num_subcores=NS)

    st = jax.ShapeDtypeStruct((R, CC), jnp.int32)
    scratch = (
        [pltpu.VMEM((P, CC), jnp.int32)] * 3
        + [pltpu.VMEM((P, CC), jnp.int32)] * 3
        + [pltpu.VMEM((P, CC), jnp.int32)] * 12
        + [pltpu.VMEM((P,), jnp.int32)] * 12
        + [pltpu.VMEM((P,), jnp.float32)] * 12
        + [pltpu.SemaphoreType.DMA] * 18
        + [pltpu.VMEM((CI + CF, NPX // NS), jnp.float32)]
    )

    @functools.partial(
        pl.kernel,
        out_type=(st, st, st, st),
        mesh=mesh,
        scratch_types=scratch,
        compiler_params=pltpu.CompilerParams(
            needs_layout_passes=False, use_tc_tiling_on_sc=False),
    )
    def pscan_kernel(imgs_cm, flows_cm, out, st0, st1, st2, *scr):
        cur = scr[0:3]
        oro = scr[3:6]
        taps = [scr[6 + 4 * q:10 + 4 * q] for q in range(3)]
        ib = [scr[18 + 4 * q:22 + 4 * q] for q in range(3)]
        wb = [scr[30 + 4 * q:34 + 4 * q] for q in range(3)]
        semc = scr[42:45]
        semo = scr[45:48]
        semg = [scr[48 + 4 * q:52 + 4 * q] for q in range(3)]
        stage = scr[60]

        core = lax.axis_index("c")
        sid = lax.axis_index("s")
        rsd = ((st0, st1), (st1, st2), (st2, out))
        TPX = NPX // NS

        @pl.loop(0, L)
        def _pack(l):
            for ch in range(CI):
                pltpu.async_copy(
                    imgs_cm.at[(core * L + l) * CI + ch, pl.ds(sid * TPX, TPX)],
                    stage.at[ch], semc[0])
            for ch in range(CF):
                pltpu.async_copy(
                    flows_cm.at[(core * L + l) * CF + ch, pl.ds(sid * TPX, TPX)],
                    stage.at[CI + ch], semc[0])
            for ch in range(CI + CF):
                pltpu.make_async_copy(imgs_cm.at[0, pl.ds(0, TPX)],
                                      stage.at[ch], semc[0]).wait()
            rowbase = (core * L + l) * NPX + sid * TPX
            for c in range(TPX // P):
                q = c % 3
                if c >= 3:
                    pltpu.make_async_copy(st0.at[pl.ds(0, P)],
                                          oro[q], semo[q]).wait()

                @plsc.parallel_loop(0, GROUPS)
                def _pk(j, c=c, q=q):
                    lane = lax.iota(jnp.int32, LN)
                    rows = j * LN + lane
                    off = c * P + j * LN
                    for cw in range(CIW):
                        lo = stage[2 * cw, pl.ds(off, LN)]
                        hi = stage[2 * cw + 1, pl.ds(off, LN)]
                        w = plsc.bitcast(
                            plsc.pack(lo, hi, format=plsc.PackFormat.INTERLEAVED),
                            jnp.int32)
                        plsc.store_scatter(
                            oro[q], [rows, jnp.full((LN,), cw, jnp.int32)], w)
                    for fc in range(CF):
                        fv = plsc.bitcast(stage[CI + fc, pl.ds(off, LN)],
                                          jnp.int32)
                        plsc.store_scatter(
                            oro[q], [rows, jnp.full((LN,), CIW + fc, jnp.int32)],
                            fv)

                pltpu.async_copy(oro[q], st0.at[pl.ds(rowbase + c * P, P)],
                                 semo[q])
            for q in range(3):
                pltpu.make_async_copy(st0.at[pl.ds(0, P)],
                                      oro[q], semo[q]).wait()

        plsc.subcore_barrier()

        @pl.loop(0, 3)
        def _round(rnd):
            s = jnp.where(rnd == 0, SS[0], jnp.where(rnd == 1, SS[1], SS[2]))
            pw = jnp.where(rnd == 0, PWS[0], jnp.where(rnd == 1, PWS[1], PWS[2]))
            nbody = jnp.where(rnd == 0, NBODYS[0],
                              jnp.where(rnd == 1, NBODYS[1], NBODYS[2]))

            def fire_src(rc, buf, sem):
                for r, (sr, _) in enumerate(rsd):
                    @pl.when(rnd == r)
                    def _(sr=sr):
                        pltpu.async_copy(sr.at[pl.ds(rc, P)], buf, sem)

            def fire_gather(idxref, buf, sem):
                for r, (sr, _) in enumerate(rsd):
                    @pl.when(rnd == r)
                    def _(sr=sr):
                        pltpu.async_copy(sr.at[idxref], buf, sem)

            def fire_store(buf, rc, sem):
                for r, (_, dsr) in enumerate(rsd):
                    @pl.when(rnd == r)
                    def _(dsr=dsr):
                        pltpu.async_copy(buf, dsr.at[pl.ds(rc, P)], sem)

            def wait(buf, sem):
                pltpu.make_async_copy(st0.at[pl.ds(0, P)], buf, sem).wait()

            @pl.loop(0, s)
            def _prefix(l):
                base = (core * L + l) * NPX + sid * (NPX // NS)
                for half in range(2):
                    for t in range(4):
                        fire_src(base + (half * 4 + t) * P, taps[0][t],
                                 semg[0][t])
                    for t in range(4):
                        wait(taps[0][t], semg[0][t])
                        fire_store(taps[0][t], base + (half * 4 + t) * P,
                                   semg[0][t])
                    for t in range(4):
                        wait(taps[0][t], semg[0][t])

            def coords(k):
                g = k * NS + sid
                sl = g >> 7
                p0 = (g - sl * CPS) * P
                slice_cur = core * L + sl + s
                return slice_cur * NPX + p0, (slice_cur - s) * NPX, p0

            def fire_cur(k, q):
                rc, _, _ = coords(k)
                fire_src(rc, cur[q], semc[q])

            def idx_pass(k, q):
                _, prev_base, p0 = coords(k)

                @plsc.parallel_loop(0, GROUPS)
                def _idx(j):
                    lane = lax.iota(jnp.int32, LN)
                    loc = j * LN + lane
                    pix = p0 + loc
                    wi = pix & (W - 1)
                    hi = pix >> 7
                    fx = plsc.bitcast(
                        plsc.load_gather(
                            cur[q], [loc, jnp.full((LN,), CIW, jnp.int32)]),
                        jnp.float32)
                    fy = plsc.bitcast(
                        plsc.load_gather(
                            cur[q], [loc, jnp.full((LN,), CIW + 1, jnp.int32)]),
                        jnp.float32)
                    gx = wi.astype(jnp.float32) * (2.0 / W) + (1.0 / W - 1.0)
                    gy = hi.astype(jnp.float32) * (2.0 / H) + (1.0 / H - 1.0)
                    tx = gx + fx + 1.0
                    tx = tx - 2.0 * _floorf(tx * 0.5)
                    rx = tx * (W * 0.5) - 0.5
                    ry = (gy + fy + 1.0) * (H * 0.5) - 0.5
                    x0 = _floorf(rx)
                    y0 = _floorf(ry)
                    wx1 = rx - x0
                    wx0 = 1.0 - wx1
                    wy1 = ry - y0
                    wy0 = 1.0 - wy1
                    ix0 = x0.astype(jnp.int32)
                    iy0 = y0.astype(jnp.int32)
                    ix1 = ix0 + 1
                    iy1 = iy0 + 1

                    def tap(iy, ix, wgt, t):
                        valid = (ix >= 0) & (ix < W) & (iy >= 0) & (iy < H)
                        idx = (prev_base
                               + jnp.clip(iy, 0, H - 1) * W
                               + jnp.clip(ix, 0, W - 1))
                        ib[q][t][pl.ds(j * LN, LN)] = idx
                        wb[q][t][pl.ds(j * LN, LN)] = jnp.where(valid, wgt, 0.0)

                    tap(iy0, ix0, wx0 * wy0, 0)
                    tap(iy0, ix1, wx1 * wy0, 1)
                    tap(iy1, ix0, wx0 * wy1, 2)
                    tap(iy1, ix1, wx1 * wy1, 3)

            def blend(k, q):
                rc, _, _ = coords(k)
                for t in range(4):
                    wait(taps[q][t], semg[q][t])

                @plsc.parallel_loop(0, GROUPS, unroll=2)
                def _blend(j):
                    lane = lax.iota(jnp.int32, LN)
                    rows = j * LN + lane
                    wv0 = wb[q][0][pl.ds(j * LN, LN)]
                    wv1 = wb[q][1][pl.ds(j * LN, LN)]
                    wv2 = wb[q][2][pl.ds(j * LN, LN)]
                    wv3 = wb[q][3][pl.ds(j * LN, LN)]
                    wp0 = plsc.pack(wv0, wv0, format=plsc.PackFormat.INTERLEAVED)
                    wp1 = plsc.pack(wv1, wv1, format=plsc.PackFormat.INTERLEAVED)
                    wp2 = plsc.pack(wv2, wv2, format=plsc.PackFormat.INTERLEAVED)
                    wp3 = plsc.pack(wv3, wv3, format=plsc.PackFormat.INTERLEAVED)
                    for c in range(CIW):
                        cc = jnp.full((LN,), c, jnp.int32)
                        g0 = plsc.bitcast(
                            plsc.load_gather(taps[q][0], [rows, cc]), jnp.bfloat16)
                        g1 = plsc.bitcast(
                            plsc.load_gather(taps[q][1], [rows, cc]), jnp.bfloat16)
                        g2 = plsc.bitcast(
                            plsc.load_gather(taps[q][2], [rows, cc]), jnp.bfloat16)
                        g3 = plsc.bitcast(
                            plsc.load_gather(taps[q][3], [rows, cc]), jnp.bfloat16)
                        res = plsc.bitcast(
                            plsc.load_gather(cur[q], [rows, cc]), jnp.bfloat16)
                        acc = (res + (wp0 * g0 + wp1 * g1)) + (wp2 * g2 + wp3 * g3)
                        plsc.store_scatter(oro[q], [rows, cc],
                                           plsc.bitcast(acc, jnp.int32))
                    for c in (CIW, CIW + 1):
                        cc = jnp.full((LN,), c, jnp.int32)
                        g0 = plsc.bitcast(
                            plsc.load_gather(taps[q][0], [rows, cc]), jnp.float32)
                        g1 = plsc.bitcast(
                            plsc.load_gather(taps[q][1], [rows, cc]), jnp.float32)
                        g2 = plsc.bitcast(
                            plsc.load_gather(taps[q][2], [rows, cc]), jnp.float32)
                        g3 = plsc.bitcast(
                            plsc.load_gather(taps[q][3], [rows, cc]), jnp.float32)
                        res = plsc.bitcast(
                            plsc.load_gather(cur[q], [rows, cc]), jnp.float32)
                        acc = res + (wv0 * g0 + wv1 * g1) + (wv2 * g2 + wv3 * g3)
                        plsc.store_scatter(oro[q], [rows, cc],
                                           plsc.bitcast(acc, jnp.int32))

                fire_store(oro[q], rc, semo[q])

            fire_cur(0, 0)

            @pl.loop(0, nbody)
            def _outer(kk):
                for u in range(3):
                    k = kk * 3 + u
                    p1 = (u + 1) % 3
                    p2 = (u + 2) % 3

                    @pl.when(k < pw)
                    def _(k=k, u=u):
                        wait(cur[u], semc[u])
                        idx_pass(k, u)
                        for t in range(4):
                            fire_gather(ib[u][t], taps[u][t], semg[u][t])

                    @pl.when((k >= 2) & (k <= pw + 1))
                    def _(k=k, p1=p1):
                        wait(oro[p1], semo[p1])

                    @pl.when(k <= pw - 2)
                    def _(k=k, p1=p1):
                        fire_cur(k + 1, p1)

                    @pl.when((k >= 1) & (k <= pw))
                    def _(k=k, p2=p2):
                        blend(k - 1, p2)

            plsc.subcore_barrier()

    return pscan_kernel


_KERNEL = _make_kernel()


def kernel(flows, images):
    fl = flows.astype(jnp.float32)
    im = images.astype(jnp.float32)
    state, _, _, _ = _KERNEL(im.reshape(B * L * CI, NPX),
                             fl.reshape(B * L * CF, NPX))
    out_bf = lax.bitcast_convert_type(
        state.reshape(B, L, NPX, CC)[..., :CIW], jnp.bfloat16)
    out = out_bf.reshape(B, L, NPX, CI).astype(jnp.float32)
    return jnp.transpose(out, (0, 1, 3, 2)).reshape(B, L, CI, H, W)


# speedup vs baseline: 2.9653x; 1.4602x over previous
import functools

import jax
import jax.numpy as jnp
from jax import lax
from jax.experimental import pallas as pl
from jax.experimental.pallas import tpu as pltpu
from jax.experimental.pallas import tpu_sc as plsc

B, L, H, W = 2, 8, 128, 128
CI, CF = 32, 2
CIW = CI // 2
CC = 24
NPX = H * W
R = B * L * NPX
NC, NS, LN = 2, 16, 16
P = 128
GROUPS = P // LN
CPS = NPX // P
PWS = (56, 48, 32)
NBODYS = (20, 17, 12)
SS = (1, 2, 4)


def _floorf(x):
    i = x.astype(jnp.int32)
    f = i.astype(jnp.float32)
    return jnp.where(f > x, f - 1.0, f)


def _make_kernel():
    mesh = plsc.VectorSubcoreMesh(
        core_axis_name="c", subcore_axis_name="s",
        num_cores=NC, num_subcores=NS)

    st = jax.ShapeDtypeStruct((R, CC), jnp.int32)
    oimg = jax.ShapeDtypeStruct((B * L * CI, NPX), jnp.float32)
    scratch = (
        [pltpu.VMEM((P, CC), jnp.int32)] * 3
        + [pltpu.VMEM((P, CC), jnp.int32)] * 3
        + [pltpu.VMEM((P, CC), jnp.int32)] * 12
        + [pltpu.VMEM((P,), jnp.int32)] * 12
        + [pltpu.VMEM((P,), jnp.float32)] * 12
        + [pltpu.SemaphoreType.DMA] * 18
        + [pltpu.VMEM((CI + CF, NPX // NS), jnp.float32)]
        + [pltpu.VMEM((CI, P), jnp.float32)] * 3
    )

    @functools.partial(
        pl.kernel,
        out_type=(oimg, st, st, st, st),
        mesh=mesh,
        scratch_types=scratch,
        compiler_params=pltpu.CompilerParams(
            needs_layout_passes=False, use_tc_tiling_on_sc=False),
    )
    def pscan_kernel(imgs_cm, flows_cm, out_img, st0, st1, st2, st3, *scr):
        cur = scr[0:3]
        oro = scr[3:6]
        taps = [scr[6 + 4 * q:10 + 4 * q] for q in range(3)]
        ib = [scr[18 + 4 * q:22 + 4 * q] for q in range(3)]
        wb = [scr[30 + 4 * q:34 + 4 * q] for q in range(3)]
        semc = scr[42:45]
        semo = scr[45:48]
        semg = [scr[48 + 4 * q:52 + 4 * q] for q in range(3)]
        stage = scr[60]
        planes = scr[61:64]

        core = lax.axis_index("c")
        sid = lax.axis_index("s")
        rsd = ((st0, st1), (st1, st2), (st2, st3))
        TPX = NPX // NS

        @pl.loop(0, L)
        def _pack(l):
            for ch in range(CI):
                pltpu.async_copy(
                    imgs_cm.at[(core * L + l) * CI + ch, pl.ds(sid * TPX, TPX)],
                    stage.at[ch], semc[0])
            for ch in range(CF):
                pltpu.async_copy(
                    flows_cm.at[(core * L + l) * CF + ch, pl.ds(sid * TPX, TPX)],
                    stage.at[CI + ch], semc[0])
            for ch in range(CI + CF):
                pltpu.make_async_copy(imgs_cm.at[0, pl.ds(0, TPX)],
                                      stage.at[ch], semc[0]).wait()
            rowbase = (core * L + l) * NPX + sid * TPX
            for c in range(TPX // P):
                q = c % 3
                if c >= 3:
                    pltpu.make_async_copy(st0.at[pl.ds(0, P)],
                                          oro[q], semo[q]).wait()

                @plsc.parallel_loop(0, GROUPS)
                def _pk(j, c=c, q=q):
                    lane = lax.iota(jnp.int32, LN)
                    rows = j * LN + lane
                    off = c * P + j * LN

                    @pl.loop(0, CIW)
                    def _pw(cw, rows=rows, off=off, q=q):
                        lo = stage[2 * cw, pl.ds(off, LN)]
                        hi = stage[2 * cw + 1, pl.ds(off, LN)]
                        w = plsc.bitcast(
                            plsc.pack(lo, hi, format=plsc.PackFormat.INTERLEAVED),
                            jnp.int32)
                        plsc.store_scatter(
                            oro[q], [rows, jnp.full((LN,), 0, jnp.int32) + cw], w)
                    for fc in range(CF):
                        fv = plsc.bitcast(stage[CI + fc, pl.ds(off, LN)],
                                          jnp.int32)
                        plsc.store_scatter(
                            oro[q], [rows, jnp.full((LN,), CIW + fc, jnp.int32)],
                            fv)

                pltpu.async_copy(oro[q], st0.at[pl.ds(rowbase + c * P, P)],
                                 semo[q])
            for q in range(3):
                pltpu.make_async_copy(st0.at[pl.ds(0, P)],
                                      oro[q], semo[q]).wait()

        plsc.subcore_barrier()

        @pl.loop(0, 3)
        def _round(rnd):
            s = jnp.where(rnd == 0, SS[0], jnp.where(rnd == 1, SS[1], SS[2]))
            pw = jnp.where(rnd == 0, PWS[0], jnp.where(rnd == 1, PWS[1], PWS[2]))
            nbody = jnp.where(rnd == 0, NBODYS[0],
                              jnp.where(rnd == 1, NBODYS[1], NBODYS[2]))

            def fire_src(rc, buf, sem):
                for r, (sr, _) in enumerate(rsd):
                    @pl.when(rnd == r)
                    def _(sr=sr):
                        pltpu.async_copy(sr.at[pl.ds(rc, P)], buf, sem)

            def fire_gather(idxref, buf, sem):
                for r, (sr, _) in enumerate(rsd):
                    @pl.when(rnd == r)
                    def _(sr=sr):
                        pltpu.async_copy(sr.at[idxref], buf, sem)

            def fire_store(buf, rc, sem):
                for r, (_, dsr) in enumerate(rsd):
                    @pl.when(rnd == r)
                    def _(dsr=dsr):
                        pltpu.async_copy(buf, dsr.at[pl.ds(rc, P)], sem)

            def wait(buf, sem):
                pltpu.make_async_copy(st0.at[pl.ds(0, P)], buf, sem).wait()

            @pl.loop(0, s)
            def _prefix(l):
                base = (core * L + l) * NPX + sid * (NPX // NS)
                for half in range(2):
                    for t in range(4):
                        fire_src(base + (half * 4 + t) * P, taps[0][t],
                                 semg[0][t])
                    for t in range(4):
                        wait(taps[0][t], semg[0][t])
                        fire_store(taps[0][t], base + (half * 4 + t) * P,
                                   semg[0][t])
                    for t in range(4):
                        wait(taps[0][t], semg[0][t])

            def coords(k):
                g = k * NS + sid
                sl = g >> 7
                p0 = (g - sl * CPS) * P
                slice_cur = core * L + sl + s
                return slice_cur * NPX + p0, (slice_cur - s) * NPX, p0

            def fire_cur(k, q):
                rc, _, _ = coords(k)
                fire_src(rc, cur[q], semc[q])

            def idx_pass(k, q):
                _, prev_base, p0 = coords(k)

                @plsc.parallel_loop(0, GROUPS)
                def _idx(j):
                    lane = lax.iota(jnp.int32, LN)
                    loc = j * LN + lane
                    pix = p0 + loc
                    wi = pix & (W - 1)
                    hi = pix >> 7
                    fx = plsc.bitcast(
                        plsc.load_gather(
                            cur[q], [loc, jnp.full((LN,), CIW, jnp.int32)]),
                        jnp.float32)
                    fy = plsc.bitcast(
                        plsc.load_gather(
                            cur[q], [loc, jnp.full((LN,), CIW + 1, jnp.int32)]),
                        jnp.float32)
                    gx = wi.astype(jnp.float32) * (2.0 / W) + (1.0 / W - 1.0)
                    gy = hi.astype(jnp.float32) * (2.0 / H) + (1.0 / H - 1.0)
                    tx = gx + fx + 1.0
                    tx = tx - 2.0 * _floorf(tx * 0.5)
                    rx = tx * (W * 0.5) - 0.5
                    ry = (gy + fy + 1.0) * (H * 0.5) - 0.5
                    x0 = _floorf(rx)
                    y0 = _floorf(ry)
                    wx1 = rx - x0
                    wx0 = 1.0 - wx1
                    wy1 = ry - y0
                    wy0 = 1.0 - wy1
                    ix0 = x0.astype(jnp.int32)
                    iy0 = y0.astype(jnp.int32)
                    ix1 = ix0 + 1
                    iy1 = iy0 + 1

                    def tap(iy, ix, wgt, t):
                        valid = (ix >= 0) & (ix < W) & (iy >= 0) & (iy < H)
                        idx = (prev_base
                               + jnp.clip(iy, 0, H - 1) * W
                               + jnp.clip(ix, 0, W - 1))
                        ib[q][t][pl.ds(j * LN, LN)] = idx
                        wb[q][t][pl.ds(j * LN, LN)] = jnp.where(valid, wgt, 0.0)

                    tap(iy0, ix0, wx0 * wy0, 0)
                    tap(iy0, ix1, wx1 * wy0, 1)
                    tap(iy1, ix0, wx0 * wy1, 2)
                    tap(iy1, ix1, wx1 * wy1, 3)

            def blend(k, q):
                rc, _, _ = coords(k)
                for t in range(4):
                    wait(taps[q][t], semg[q][t])

                @plsc.parallel_loop(0, GROUPS)
                def _blend(j):
                    lane = lax.iota(jnp.int32, LN)
                    rows = j * LN + lane
                    wv0 = wb[q][0][pl.ds(j * LN, LN)]
                    wv1 = wb[q][1][pl.ds(j * LN, LN)]
                    wv2 = wb[q][2][pl.ds(j * LN, LN)]
                    wv3 = wb[q][3][pl.ds(j * LN, LN)]
                    wp0 = plsc.pack(wv0, wv0, format=plsc.PackFormat.INTERLEAVED)
                    wp1 = plsc.pack(wv1, wv1, format=plsc.PackFormat.INTERLEAVED)
                    wp2 = plsc.pack(wv2, wv2, format=plsc.PackFormat.INTERLEAVED)
                    wp3 = plsc.pack(wv3, wv3, format=plsc.PackFormat.INTERLEAVED)
                    for c in range(CIW):
                        cc = jnp.full((LN,), c, jnp.int32)
                        g0 = plsc.bitcast(
                            plsc.load_gather(taps[q][0], [rows, cc]), jnp.bfloat16)
                        g1 = plsc.bitcast(
                            plsc.load_gather(taps[q][1], [rows, cc]), jnp.bfloat16)
                        g2 = plsc.bitcast(
                            plsc.load_gather(taps[q][2], [rows, cc]), jnp.bfloat16)
                        g3 = plsc.bitcast(
                            plsc.load_gather(taps[q][3], [rows, cc]), jnp.bfloat16)
                        res = plsc.bitcast(
                            plsc.load_gather(cur[q], [rows, cc]), jnp.bfloat16)
                        acc = (res + (wp0 * g0 + wp1 * g1)) + (wp2 * g2 + wp3 * g3)
                        plsc.store_scatter(oro[q], [rows, cc],
                                           plsc.bitcast(acc, jnp.int32))
                    for c in (CIW, CIW + 1):
                        cc = jnp.full((LN,), c, jnp.int32)
                        g0 = plsc.bitcast(
                            plsc.load_gather(taps[q][0], [rows, cc]), jnp.float32)
                        g1 = plsc.bitcast(
                            plsc.load_gather(taps[q][1], [rows, cc]), jnp.float32)
                        g2 = plsc.bitcast(
                            plsc.load_gather(taps[q][2], [rows, cc]), jnp.float32)
                        g3 = plsc.bitcast(
                            plsc.load_gather(taps[q][3], [rows, cc]), jnp.float32)
                        res = plsc.bitcast(
                            plsc.load_gather(cur[q], [rows, cc]), jnp.float32)
                        acc = res + (wv0 * g0 + wv1 * g1) + (wv2 * g2 + wv3 * g3)
                        plsc.store_scatter(oro[q], [rows, cc],
                                           plsc.bitcast(acc, jnp.int32))

                fire_store(oro[q], rc, semo[q])

            fire_cur(0, 0)

            @pl.loop(0, nbody)
            def _outer(kk):
                for u in range(3):
                    k = kk * 3 + u
                    p1 = (u + 1) % 3
                    p2 = (u + 2) % 3

                    @pl.when(k < pw)
                    def _(k=k, u=u):
                        wait(cur[u], semc[u])
                        idx_pass(k, u)
                        for t in range(4):
                            fire_gather(ib[u][t], taps[u][t], semg[u][t])

                    @pl.when((k >= 2) & (k <= pw + 1))
                    def _(k=k, p1=p1):
                        wait(oro[p1], semo[p1])

                    @pl.when(k <= pw - 2)
                    def _(k=k, p1=p1):
                        fire_cur(k + 1, p1)

                    @pl.when((k >= 1) & (k <= pw))
                    def _(k=k, p2=p2):
                        blend(k - 1, p2)

            plsc.subcore_barrier()

        @pl.loop(0, L)
        def _unpack(l):
            rowbase = (core * L + l) * NPX + sid * TPX
            pltpu.async_copy(st3.at[pl.ds(rowbase, P)], cur[0], semc[0])
            for c in range(TPX // P):
                q = c % 3
                pltpu.make_async_copy(st0.at[pl.ds(0, P)],
                                      cur[q], semc[q]).wait()
                if c + 1 < TPX // P:
                    qn = (c + 1) % 3
                    pltpu.async_copy(st3.at[pl.ds(rowbase + (c + 1) * P, P)],
                                     cur[qn], semc[qn])
                if c >= 3:
                    for ch in range(CI):
                        pltpu.make_async_copy(
                            imgs_cm.at[0, pl.ds(0, P)],
                            planes[q].at[ch], semo[q]).wait()

                @plsc.parallel_loop(0, GROUPS)
                def _up(j, q=q):
                    lane = lax.iota(jnp.int32, LN)
                    rows = j * LN + lane

                    @pl.loop(0, CIW)
                    def _uw(cw, rows=rows, j=j, q=q):
                        v = plsc.bitcast(
                            plsc.load_gather(
                                cur[q], [rows, jnp.full((LN,), 0, jnp.int32) + cw]),
                            jnp.bfloat16)
                        lo, hi = plsc.unpack(
                            v, format=plsc.PackFormat.INTERLEAVED)
                        planes[q][2 * cw, pl.ds(j * LN, LN)] = lo
                        planes[q][2 * cw + 1, pl.ds(j * LN, LN)] = hi
                for ch in range(CI):
                    pltpu.async_copy(
                        planes[q].at[ch],
                        out_img.at[(core * L + l) * CI + ch,
                                   pl.ds(sid * TPX + c * P, P)],
                        semo[q])
            for c in range(5, 8):
                q = c % 3
                for ch in range(CI):
                    pltpu.make_async_copy(imgs_cm.at[0, pl.ds(0, P)],
                                          planes[q].at[ch], semo[q]).wait()

    return pscan_kernel


_KERNEL = _make_kernel()


def kernel(flows, images):
    fl = flows.astype(jnp.float32)
    im = images.astype(jnp.float32)
    out, _, _, _, _ = _KERNEL(im.reshape(B * L * CI, NPX),
                              fl.reshape(B * L * CF, NPX))
    return out.reshape(B, L, CI, H, W)
